# baseline (speedup 1.0000x reference)
import functools

import jax
import jax.numpy as jnp
from jax import lax
from jax.experimental import pallas as pl
from jax.experimental.pallas import tpu as pltpu
from jax.experimental.pallas import tpu_sc as plsc


def _sc_gather(user_id, item_id, category, W_user, W_item, W_cat):
    info = plsc.get_sparse_core_info()
    NC, NS = info.num_cores, info.num_subcores
    NW = NC * NS
    B = user_id.shape[0]
    DU = W_user.shape[1]
    DI = W_item.shape[1]
    DC = W_cat.shape[1]
    assert B % NW == 0
    b_per_w = B // NW
    mesh = plsc.VectorSubcoreMesh(core_axis_name="c", subcore_axis_name="s")

    @functools.partial(
        pl.kernel,
        mesh=mesh,
        compiler_params=pltpu.CompilerParams(use_tc_tiling_on_sc=False),
        out_type=(
            jax.ShapeDtypeStruct((B, DU), jnp.float32),
            jax.ShapeDtypeStruct((B, DI), jnp.float32),
            jax.ShapeDtypeStruct((B, DC), jnp.float32),
        ),
        scratch_types=[
            pltpu.VMEM((b_per_w,), jnp.int32),
            pltpu.VMEM((b_per_w,), jnp.int32),
            pltpu.VMEM((b_per_w,), jnp.int32),
            pltpu.VMEM((b_per_w, DU), jnp.float32),
            pltpu.VMEM((b_per_w, DI), jnp.float32),
            pltpu.VMEM((b_per_w, DC), jnp.float32),
            pltpu.SemaphoreType.DMA,
        ],
    )
    def k(uid_hbm, iid_hbm, cid_hbm, wu_hbm, wi_hbm, wc_hbm,
          eu_hbm, ei_hbm, ec_hbm,
          uidx, iidx, cidx, urows, irows, crows, sem):
        wid = lax.axis_index("s") * NC + lax.axis_index("c")
        row0 = wid * b_per_w
        pltpu.sync_copy(uid_hbm.at[pl.ds(row0, b_per_w)], uidx)
        pltpu.sync_copy(iid_hbm.at[pl.ds(row0, b_per_w)], iidx)
        pltpu.sync_copy(cid_hbm.at[pl.ds(row0, b_per_w)], cidx)
        cu = pltpu.async_copy(wu_hbm.at[uidx], urows, sem)
        ci = pltpu.async_copy(wi_hbm.at[iidx], irows, sem)
        cc = pltpu.async_copy(wc_hbm.at[cidx], crows, sem)
        cu.wait()
        ci.wait()
        cc.wait()
        pltpu.sync_copy(urows, eu_hbm.at[pl.ds(row0, b_per_w)])
        pltpu.sync_copy(irows, ei_hbm.at[pl.ds(row0, b_per_w)])
        pltpu.sync_copy(crows, ec_hbm.at[pl.ds(row0, b_per_w)])

    return k(user_id, item_id, category, W_user, W_item, W_cat)


def _tc_project(base, e_user, e_item, e_cat, W_proj, b_proj):
    B, DB = base.shape
    DU = e_user.shape[1]
    DI = e_item.shape[1]
    DC = e_cat.shape[1]
    N = W_proj.shape[1]
    K = W_proj.shape[0]
    BLK = 512
    grid = (B // BLK,)

    def body(base_ref, eu_ref, ei_ref, ec_ref, wp_ref, b_ref, out_ref):
        acc = jnp.dot(base_ref[...], wp_ref[0:DB, :],
                      preferred_element_type=jnp.float32)
        acc += jnp.dot(eu_ref[...], wp_ref[DB:DB + DU, :],
                       preferred_element_type=jnp.float32)
        acc += jnp.dot(ei_ref[...], wp_ref[DB + DU:DB + DU + DI, :],
                       preferred_element_type=jnp.float32)
        acc += jnp.dot(ec_ref[...], wp_ref[DB + DU + DI:K, :],
                       preferred_element_type=jnp.float32)
        out_ref[...] = acc + b_ref[...]

    return pl.pallas_call(
        body,
        grid=grid,
        in_specs=[
            pl.BlockSpec((BLK, DB), lambda i: (i, 0)),
            pl.BlockSpec((BLK, DU), lambda i: (i, 0)),
            pl.BlockSpec((BLK, DI), lambda i: (i, 0)),
            pl.BlockSpec((BLK, DC), lambda i: (i, 0)),
            pl.BlockSpec((K, N), lambda i: (0, 0)),
            pl.BlockSpec((1, N), lambda i: (0, 0)),
        ],
        out_specs=pl.BlockSpec((BLK, N), lambda i: (i, 0)),
        out_shape=jax.ShapeDtypeStruct((B, N), jnp.float32),
    )(base, e_user, e_item, e_cat, W_proj, b_proj.reshape(1, N))


def kernel(base, user_id, item_id, category, W_user, W_item, W_cat, W_proj, b_proj):
    user_id = user_id.astype(jnp.int32)
    item_id = item_id.astype(jnp.int32)
    category = category.astype(jnp.int32)
    e_user, e_item, e_cat = _sc_gather(user_id, item_id, category,
                                       W_user, W_item, W_cat)
    return _tc_project(base, e_user, e_item, e_cat, W_proj, b_proj)
